# baseline (speedup 1.0000x reference)
import jax
import jax.numpy as jnp
from jax.experimental import pallas as pl
from jax.experimental.pallas import tpu as pltpu

N = 10000
D_IN = 128
D_OUT = 32

ROW_BLOCK = 400


def _xw_kernel(x_ref, w_ref, o_ref):
    o_ref[...] = jax.lax.dot_general(
        x_ref[...], w_ref[...],
        dimension_numbers=(((1,), (0,)), ((), ())),
        preferred_element_type=jnp.float32,
    )


def _spmm_relu_kernel(adj_ref, xw_ref, o_ref):
    acc = jax.lax.dot_general(
        adj_ref[...], xw_ref[...],
        dimension_numbers=(((1,), (0,)), ((), ())),
        preferred_element_type=jnp.float32,
    )
    o_ref[...] = jnp.maximum(acc, 0.0)


def kernel(inputs, adj_norm, weight):
    xw = pl.pallas_call(
        _xw_kernel,
        out_shape=jax.ShapeDtypeStruct((N, D_OUT), jnp.float32),
    )(inputs, weight)

    grid = (N // ROW_BLOCK,)
    out = pl.pallas_call(
        _spmm_relu_kernel,
        grid=grid,
        in_specs=[
            pl.BlockSpec((ROW_BLOCK, N), lambda i: (i, 0)),
            pl.BlockSpec((N, D_OUT), lambda i: (0, 0)),
        ],
        out_specs=pl.BlockSpec((ROW_BLOCK, D_OUT), lambda i: (i, 0)),
        out_shape=jax.ShapeDtypeStruct((N, D_OUT), jnp.float32),
        compiler_params=pltpu.CompilerParams(
            dimension_semantics=("arbitrary",),
        ),
    )(adj_norm, xw)
    return out


# speedup vs baseline: 1.0455x; 1.0455x over previous
import jax
import jax.numpy as jnp
from jax.experimental import pallas as pl
from jax.experimental.pallas import tpu as pltpu

N = 10000
D_IN = 128
D_OUT = 32

ROW_BLOCK = 400


def _fused_kernel(x_ref, w_ref, adj_ref, o_ref, xw_ref):
    @pl.when(pl.program_id(0) == 0)
    def _():
        xw_ref[...] = jax.lax.dot_general(
            x_ref[...], w_ref[...],
            dimension_numbers=(((1,), (0,)), ((), ())),
            preferred_element_type=jnp.float32,
        )

    acc = jax.lax.dot_general(
        adj_ref[...], xw_ref[...],
        dimension_numbers=(((1,), (0,)), ((), ())),
        preferred_element_type=jnp.float32,
    )
    o_ref[...] = jnp.maximum(acc, 0.0)


def kernel(inputs, adj_norm, weight):
    grid = (N // ROW_BLOCK,)
    out = pl.pallas_call(
        _fused_kernel,
        grid=grid,
        in_specs=[
            pl.BlockSpec((N, D_IN), lambda i: (0, 0)),
            pl.BlockSpec((D_IN, D_OUT), lambda i: (0, 0)),
            pl.BlockSpec((ROW_BLOCK, N), lambda i: (i, 0)),
        ],
        out_specs=pl.BlockSpec((ROW_BLOCK, D_OUT), lambda i: (i, 0)),
        out_shape=jax.ShapeDtypeStruct((N, D_OUT), jnp.float32),
        scratch_shapes=[pltpu.VMEM((N, D_OUT), jnp.float32)],
        compiler_params=pltpu.CompilerParams(
            dimension_semantics=("arbitrary",),
        ),
    )(inputs, weight, adj_norm)
    return out
